# baseline (speedup 1.0000x reference)
import functools

import jax
import jax.numpy as jnp
from jax import lax
from jax.experimental import pallas as pl
from jax.experimental.pallas import tpu as pltpu
from jax.experimental.pallas import tpu_sc as plsc

N = 10000
E = 320000
F = 128
NC = 2
NS = 16
NW = NC * NS
K = 128
CPT = 80
EPT = CPT * K
EPAD = NW * EPT
JUNK = 10239
NPAD = 10240
RPT = NPAD // NS
ZR = 128
BR = 128
G = (N + BR - 1) // BR

_mesh = plsc.VectorSubcoreMesh(core_axis_name="c", subcore_axis_name="s")


K1 = 80
NCHUNK1 = 10000 // K1

@functools.partial(
    pl.kernel,
    out_type=jax.ShapeDtypeStruct((NC, NPAD, 16), jnp.float32),
    mesh=_mesh,
    scratch_types=[
        pltpu.VMEM_SHARED((NPAD, 16), jnp.float32),
        pltpu.VMEM((K1,), jnp.int32),
        pltpu.VMEM((K1, 16), jnp.float32),
        pltpu.VMEM((RPT, 16), jnp.float32),
    ],
)
def _sc_degree(dst_hbm, out_hbm, hist, idx_v, obuf, zbuf):
    cid = lax.axis_index("c")
    sid = lax.axis_index("s")
    wid = sid * NC + cid

    z16 = jnp.zeros((16,), jnp.float32)
    io = lax.iota(jnp.int32, 16)
    v1 = jnp.where(io == 0, 1.0, 0.0).astype(jnp.float32)

    def _zb(r, carry):
        zbuf[r] = z16
        return carry
    lax.fori_loop(0, RPT, _zb, 0)

    def _ob(r, carry):
        obuf[r] = v1
        return carry
    lax.fori_loop(0, K1, _ob, 0)

    pltpu.sync_copy(zbuf, hist.at[pl.ds(sid * RPT, RPT)])
    plsc.subcore_barrier()

    def _chunk(c, carry):
        pltpu.sync_copy(dst_hbm.at[pl.ds(wid * 10000 + c * K1, K1)], idx_v)
        pltpu.sync_copy(obuf, hist.at[idx_v], add=True)
        return carry
    lax.fori_loop(0, NCHUNK1, _chunk, 0)
    plsc.subcore_barrier()

    pltpu.sync_copy(hist.at[pl.ds(sid * RPT, RPT)], zbuf)
    pltpu.sync_copy(zbuf, out_hbm.at[cid, pl.ds(sid * RPT, RPT)])


@functools.partial(
    pl.kernel,
    out_type=jax.ShapeDtypeStruct((NC, NPAD, F), jnp.float32),
    mesh=_mesh,
    scratch_types=[
        pltpu.VMEM_SHARED((NPAD, F), jnp.float32),
        pltpu.VMEM((K,), jnp.int32),
        pltpu.VMEM((K,), jnp.int32),
        pltpu.VMEM((K,), jnp.int32),
        pltpu.VMEM((K,), jnp.int32),
        pltpu.VMEM((K, F), jnp.float32),
        pltpu.VMEM((K, F), jnp.float32),
        pltpu.SemaphoreType.DMA,
        pltpu.SemaphoreType.DMA,
        pltpu.SemaphoreType.DMA,
        pltpu.SemaphoreType.DMA,
    ],
)
def _sc_spmm(xs_hbm, src_hbm, dst_hbm, out_hbm,
             acc, sidx0, sidx1, didx0, didx1, rows0, rows1,
             is0, is1, gs0, gs1):
    cid = lax.axis_index("c")
    sid = lax.axis_index("s")
    wid = sid * NC + cid
    base = wid * CPT

    z16 = jnp.zeros((16,), jnp.float32)

    def _zb(r, carry):
        for j in range(F // 16):
            rows0[r, pl.ds(j * 16, 16)] = z16
        return carry
    lax.fori_loop(0, ZR, _zb, 0)

    for t in range(RPT // ZR):
        pltpu.sync_copy(rows0, acc.at[pl.ds(sid * RPT + t * ZR, ZR)])
    plsc.subcore_barrier()

    def _fire_idx(c, sb, db, sem):
        pltpu.async_copy(src_hbm.at[base + c], sb, sem)
        pltpu.async_copy(dst_hbm.at[base + c], db, sem)

    def _wait_idx(c, sb, db, sem):
        pltpu.make_async_copy(src_hbm.at[base + c], sb, sem).wait()
        pltpu.make_async_copy(dst_hbm.at[base + c], db, sem).wait()

    _fire_idx(0, sidx0, didx0, is0)

    def _pair(t, carry):
        c0 = 2 * t
        c1 = c0 + 1
        c2 = jnp.minimum(c0 + 2, CPT - 1)
        _wait_idx(c0, sidx0, didx0, is0)
        _fire_idx(c1, sidx1, didx1, is1)
        pltpu.async_copy(xs_hbm.at[sidx0], rows0, gs0).wait()
        pltpu.sync_copy(rows0, acc.at[didx0], add=True)
        _wait_idx(c1, sidx1, didx1, is1)
        _fire_idx(c2, sidx0, didx0, is0)
        pltpu.async_copy(xs_hbm.at[sidx1], rows1, gs1).wait()
        pltpu.sync_copy(rows1, acc.at[didx1], add=True)
        return carry
    lax.fori_loop(0, CPT // 2, _pair, 0)
    _wait_idx(CPT - 1, sidx0, didx0, is0)
    plsc.subcore_barrier()

    for t in range(RPT // ZR):
        pltpu.sync_copy(acc.at[pl.ds(sid * RPT + t * ZR, ZR)], rows0)
        pltpu.sync_copy(rows0, out_hbm.at[cid, pl.ds(sid * RPT + t * ZR, ZR)])


def _dis_scale_body(deg_ref, x_ref, dis_ref, xs_ref):
    p = deg_ref[...]
    deg = p[0, :, 0:1] + p[1, :, 0:1] + 1.0
    dis = lax.rsqrt(deg)
    dis_ref[...] = dis
    xs_ref[...] = x_ref[...] * dis


def _tc_dis_scale(deg_parts, x):
    return pl.pallas_call(
        _dis_scale_body,
        grid=(G,),
        in_specs=[
            pl.BlockSpec((NC, BR, 16), lambda i: (0, i, 0)),
            pl.BlockSpec((BR, F), lambda i: (i, 0)),
        ],
        out_specs=[
            pl.BlockSpec((BR, 1), lambda i: (i, 0)),
            pl.BlockSpec((BR, F), lambda i: (i, 0)),
        ],
        out_shape=[
            jax.ShapeDtypeStruct((G * BR, 1), jnp.float32),
            jax.ShapeDtypeStruct((N, F), jnp.float32),
        ],
    )(deg_parts, x)


def _final_body(p_ref, xs_ref, dis_ref, w_ref, b_ref, o_ref):
    p = p_ref[...]
    agg = (p[0] + p[1] + xs_ref[...]) * dis_ref[...]
    o_ref[...] = lax.dot_general(
        agg, w_ref[...], (((1,), (1,)), ((), ())),
        preferred_element_type=jnp.float32) + b_ref[...]


def _tc_final(parts, xs, dis, W, b2):
    return pl.pallas_call(
        _final_body,
        grid=(G,),
        in_specs=[
            pl.BlockSpec((NC, BR, F), lambda i: (0, i, 0)),
            pl.BlockSpec((BR, F), lambda i: (i, 0)),
            pl.BlockSpec((BR, 1), lambda i: (i, 0)),
            pl.BlockSpec((F, F), lambda i: (0, 0)),
            pl.BlockSpec((1, F), lambda i: (0, 0)),
        ],
        out_specs=pl.BlockSpec((BR, F), lambda i: (i, 0)),
        out_shape=jax.ShapeDtypeStruct((N, F), jnp.float32),
    )(parts, xs, dis, W, b2)


def kernel(input_x, edge_index, W, b):
    src = edge_index[0].astype(jnp.int32)
    dst = edge_index[1].astype(jnp.int32)
    srcp = jnp.pad(src, (0, EPAD - E)).reshape(NW * CPT, K)
    dstp = jnp.pad(dst, (0, EPAD - E),
                   constant_values=JUNK).reshape(NW * CPT, K)
    deg_parts = _sc_degree(dst)
    dis, xs = _tc_dis_scale(deg_parts, input_x)
    parts = _sc_spmm(xs, srcp, dstp)
    out = _tc_final(parts, xs, dis, W, b.reshape(1, F))
    return out


# speedup vs baseline: 1.0156x; 1.0156x over previous
import functools

import jax
import jax.numpy as jnp
from jax import lax
from jax.experimental import pallas as pl
from jax.experimental.pallas import tpu as pltpu
from jax.experimental.pallas import tpu_sc as plsc

N = 10000
E = 320000
F = 128
NC = 2
NS = 16
NW = NC * NS
K = 128
CPT = 80
EPT = CPT * K
EPAD = NW * EPT
JUNK = 10239
NPAD = 10240
RPT = NPAD // NS
ZR = 128
BR = 128
G = (N + BR - 1) // BR

_mesh = plsc.VectorSubcoreMesh(core_axis_name="c", subcore_axis_name="s")


K1 = 80
NCHUNK1 = 10000 // K1

@functools.partial(
    pl.kernel,
    out_type=jax.ShapeDtypeStruct((NC, NPAD, 16), jnp.float32),
    mesh=_mesh,
    scratch_types=[
        pltpu.VMEM_SHARED((NPAD, 16), jnp.float32),
        pltpu.VMEM((K1,), jnp.int32),
        pltpu.VMEM((K1, 16), jnp.float32),
        pltpu.VMEM((RPT, 16), jnp.float32),
    ],
)
def _sc_degree(dst_hbm, out_hbm, hist, idx_v, obuf, zbuf):
    cid = lax.axis_index("c")
    sid = lax.axis_index("s")
    wid = sid * NC + cid

    z16 = jnp.zeros((16,), jnp.float32)
    io = lax.iota(jnp.int32, 16)
    v1 = jnp.where(io == 0, 1.0, 0.0).astype(jnp.float32)

    def _zb(r, carry):
        zbuf[r] = z16
        return carry
    lax.fori_loop(0, RPT, _zb, 0)

    def _ob(r, carry):
        obuf[r] = v1
        return carry
    lax.fori_loop(0, K1, _ob, 0)

    pltpu.sync_copy(zbuf, hist.at[pl.ds(sid * RPT, RPT)])
    plsc.subcore_barrier()

    def _chunk(c, carry):
        pltpu.sync_copy(dst_hbm.at[pl.ds(wid * 10000 + c * K1, K1)], idx_v)
        pltpu.sync_copy(obuf, hist.at[idx_v], add=True)
        return carry
    lax.fori_loop(0, NCHUNK1, _chunk, 0)
    plsc.subcore_barrier()

    pltpu.sync_copy(hist.at[pl.ds(sid * RPT, RPT)], zbuf)
    pltpu.sync_copy(zbuf, out_hbm.at[cid, pl.ds(sid * RPT, RPT)])


@functools.partial(
    pl.kernel,
    out_type=jax.ShapeDtypeStruct((NC, NPAD, F), jnp.float32),
    mesh=_mesh,
    scratch_types=[
        pltpu.VMEM_SHARED((NPAD, F), jnp.float32),
        pltpu.VMEM((K,), jnp.int32),
        pltpu.VMEM((K,), jnp.int32),
        pltpu.VMEM((K, F), jnp.float32),
        pltpu.VMEM((ZR, F), jnp.float32),
        pltpu.SemaphoreType.DMA,
    ],
)
def _sc_spmm(xs_hbm, src_hbm, dst_hbm, out_hbm, acc, sidx, didx, rows, zbuf, sem):
    cid = lax.axis_index("c")
    sid = lax.axis_index("s")
    wid = sid * NC + cid

    z16 = jnp.zeros((16,), jnp.float32)

    def _zb(r, carry):
        for j in range(F // 16):
            zbuf[r, pl.ds(j * 16, 16)] = z16
        return carry
    lax.fori_loop(0, ZR, _zb, 0)

    for t in range(RPT // ZR):
        pltpu.sync_copy(zbuf, acc.at[pl.ds(sid * RPT + t * ZR, ZR)])
    plsc.subcore_barrier()

    def _chunk(c, carry):
        e0 = wid * EPT + c * K
        pltpu.sync_copy(src_hbm.at[pl.ds(e0, K)], sidx)
        pltpu.sync_copy(dst_hbm.at[pl.ds(e0, K)], didx)
        pltpu.async_copy(xs_hbm.at[sidx], rows, sem).wait()
        pltpu.sync_copy(rows, acc.at[didx], add=True)
        return carry
    lax.fori_loop(0, CPT, _chunk, 0)
    plsc.subcore_barrier()

    for t in range(RPT // ZR):
        pltpu.sync_copy(acc.at[pl.ds(sid * RPT + t * ZR, ZR)], zbuf)
        pltpu.sync_copy(zbuf, out_hbm.at[cid, pl.ds(sid * RPT + t * ZR, ZR)])


def _dis_scale_body(deg_ref, x_ref, dis_ref, xs_ref):
    p = deg_ref[...]
    deg = p[0, :, 0:1] + p[1, :, 0:1] + 1.0
    dis = lax.rsqrt(deg)
    dis_ref[...] = dis
    xs_ref[...] = x_ref[...] * dis


def _tc_dis_scale(deg_parts, x):
    return pl.pallas_call(
        _dis_scale_body,
        grid=(G,),
        in_specs=[
            pl.BlockSpec((NC, BR, 16), lambda i: (0, i, 0)),
            pl.BlockSpec((BR, F), lambda i: (i, 0)),
        ],
        out_specs=[
            pl.BlockSpec((BR, 1), lambda i: (i, 0)),
            pl.BlockSpec((BR, F), lambda i: (i, 0)),
        ],
        out_shape=[
            jax.ShapeDtypeStruct((G * BR, 1), jnp.float32),
            jax.ShapeDtypeStruct((N, F), jnp.float32),
        ],
    )(deg_parts, x)


def _final_body(p_ref, xs_ref, dis_ref, w_ref, b_ref, o_ref):
    p = p_ref[...]
    agg = (p[0] + p[1] + xs_ref[...]) * dis_ref[...]
    o_ref[...] = lax.dot_general(
        agg, w_ref[...], (((1,), (1,)), ((), ())),
        preferred_element_type=jnp.float32) + b_ref[...]


def _tc_final(parts, xs, dis, W, b2):
    return pl.pallas_call(
        _final_body,
        grid=(G,),
        in_specs=[
            pl.BlockSpec((NC, BR, F), lambda i: (0, i, 0)),
            pl.BlockSpec((BR, F), lambda i: (i, 0)),
            pl.BlockSpec((BR, 1), lambda i: (i, 0)),
            pl.BlockSpec((F, F), lambda i: (0, 0)),
            pl.BlockSpec((1, F), lambda i: (0, 0)),
        ],
        out_specs=pl.BlockSpec((BR, F), lambda i: (i, 0)),
        out_shape=jax.ShapeDtypeStruct((N, F), jnp.float32),
    )(parts, xs, dis, W, b2)


def kernel(input_x, edge_index, W, b):
    src = edge_index[0].astype(jnp.int32)
    dst = edge_index[1].astype(jnp.int32)
    srcp = jnp.pad(src, (0, EPAD - E))
    dstp = jnp.pad(dst, (0, EPAD - E), constant_values=JUNK)
    deg_parts = _sc_degree(dst)
    dis, xs = _tc_dis_scale(deg_parts, input_x)
    parts = _sc_spmm(xs, srcp, dstp)
    out = _tc_final(parts, xs, dis, W, b.reshape(1, F))
    return out


# speedup vs baseline: 1.6064x; 1.5817x over previous
import functools

import jax
import jax.numpy as jnp
from jax import lax
from jax.experimental import pallas as pl
from jax.experimental.pallas import tpu as pltpu
from jax.experimental.pallas import tpu_sc as plsc

N = 10000
E = 320000
F = 128
NC = 2
NS = 16
NW = NC * NS
K = 128
CPT = 80
EPT = CPT * K
EPAD = NW * EPT
JUNK = 10112
NPAD = 10240
RPT = NPAD // NS
ZR = 128
BR = 128
G = (N + BR - 1) // BR

_mesh = plsc.VectorSubcoreMesh(core_axis_name="c", subcore_axis_name="s")


K1 = 80
NCHUNK1 = 10000 // K1

@functools.partial(
    pl.kernel,
    out_type=jax.ShapeDtypeStruct((NC, NPAD, 16), jnp.float32),
    mesh=_mesh,
    scratch_types=[
        pltpu.VMEM_SHARED((NPAD, 16), jnp.float32),
        pltpu.VMEM((K1,), jnp.int32),
        pltpu.VMEM((K1, 16), jnp.float32),
        pltpu.VMEM((RPT, 16), jnp.float32),
    ],
)
def _sc_degree(dst_hbm, out_hbm, hist, idx_v, obuf, zbuf):
    cid = lax.axis_index("c")
    sid = lax.axis_index("s")
    wid = sid * NC + cid

    z16 = jnp.zeros((16,), jnp.float32)
    io = lax.iota(jnp.int32, 16)
    v1 = jnp.where(io == 0, 1.0, 0.0).astype(jnp.float32)

    def _zb(r, carry):
        zbuf[r] = z16
        return carry
    lax.fori_loop(0, RPT, _zb, 0)

    def _ob(r, carry):
        obuf[r] = v1
        return carry
    lax.fori_loop(0, K1, _ob, 0)

    pltpu.sync_copy(zbuf, hist.at[pl.ds(sid * RPT, RPT)])
    plsc.subcore_barrier()

    def _chunk(c, carry):
        pltpu.sync_copy(dst_hbm.at[pl.ds(wid * 10000 + c * K1, K1)], idx_v)
        pltpu.sync_copy(obuf, hist.at[idx_v], add=True)
        return carry
    lax.fori_loop(0, NCHUNK1, _chunk, 0)
    plsc.subcore_barrier()

    pltpu.sync_copy(hist.at[pl.ds(sid * RPT, RPT)], zbuf)
    pltpu.sync_copy(zbuf, out_hbm.at[cid, pl.ds(sid * RPT, RPT)])


@functools.partial(
    pl.kernel,
    out_type=jax.ShapeDtypeStruct((NC, NPAD, F), jnp.float32),
    mesh=_mesh,
    scratch_types=[
        pltpu.VMEM_SHARED((NPAD, F), jnp.float32),
        pltpu.VMEM((K,), jnp.int32),
        pltpu.VMEM((K,), jnp.int32),
        pltpu.VMEM((K, F), jnp.float32),
        pltpu.VMEM((ZR, F), jnp.float32),
        pltpu.SemaphoreType.DMA,
    ],
)
def _sc_spmm(xs_hbm, src_hbm, dst_hbm, out_hbm, acc, sidx, didx, rows, zbuf, sem):
    cid = lax.axis_index("c")
    sid = lax.axis_index("s")
    wid = sid * NC + cid

    z16 = jnp.zeros((16,), jnp.float32)

    def _zb(r, carry):
        for j in range(F // 16):
            zbuf[r, pl.ds(j * 16, 16)] = z16
        return carry
    lax.fori_loop(0, ZR, _zb, 0)

    for t in range(RPT // ZR):
        pltpu.sync_copy(zbuf, acc.at[pl.ds(sid * RPT + t * ZR, ZR)])
    plsc.subcore_barrier()

    base = wid * CPT

    def _chunk(c, carry):
        pltpu.sync_copy(src_hbm.at[base + c], sidx)
        pltpu.sync_copy(dst_hbm.at[base + c], didx)
        pltpu.async_copy(xs_hbm.at[sidx], rows, sem).wait()
        pltpu.sync_copy(rows, acc.at[didx], add=True)
        return carry
    lax.fori_loop(0, CPT, _chunk, 0)
    plsc.subcore_barrier()

    for t in range(RPT // ZR):
        pltpu.sync_copy(acc.at[pl.ds(sid * RPT + t * ZR, ZR)], zbuf)
        pltpu.sync_copy(zbuf, out_hbm.at[cid, pl.ds(sid * RPT + t * ZR, ZR)])


def _dis_scale_body(deg_ref, x_ref, dis_ref, xs_ref):
    p = deg_ref[...]
    deg = p[0, :, 0:1] + p[1, :, 0:1] + 1.0
    dis = lax.rsqrt(deg)
    dis_ref[...] = dis
    xs_ref[...] = x_ref[...] * dis


def _tc_dis_scale(deg_parts, x):
    return pl.pallas_call(
        _dis_scale_body,
        grid=(G,),
        in_specs=[
            pl.BlockSpec((NC, BR, 16), lambda i: (0, i, 0)),
            pl.BlockSpec((BR, F), lambda i: (i, 0)),
        ],
        out_specs=[
            pl.BlockSpec((BR, 1), lambda i: (i, 0)),
            pl.BlockSpec((BR, F), lambda i: (i, 0)),
        ],
        out_shape=[
            jax.ShapeDtypeStruct((G * BR, 1), jnp.float32),
            jax.ShapeDtypeStruct((N, F), jnp.float32),
        ],
    )(deg_parts, x)


def _final_body(p_ref, xs_ref, dis_ref, w_ref, b_ref, o_ref):
    p = p_ref[...]
    agg = (p[0] + p[1] + xs_ref[...]) * dis_ref[...]
    o_ref[...] = lax.dot_general(
        agg, w_ref[...], (((1,), (1,)), ((), ())),
        preferred_element_type=jnp.float32) + b_ref[...]


def _tc_final(parts, xs, dis, W, b2):
    return pl.pallas_call(
        _final_body,
        grid=(G,),
        in_specs=[
            pl.BlockSpec((NC, BR, F), lambda i: (0, i, 0)),
            pl.BlockSpec((BR, F), lambda i: (i, 0)),
            pl.BlockSpec((BR, 1), lambda i: (i, 0)),
            pl.BlockSpec((F, F), lambda i: (0, 0)),
            pl.BlockSpec((1, F), lambda i: (0, 0)),
        ],
        out_specs=pl.BlockSpec((BR, F), lambda i: (i, 0)),
        out_shape=jax.ShapeDtypeStruct((N, F), jnp.float32),
    )(parts, xs, dis, W, b2)


def kernel(input_x, edge_index, W, b):
    src = edge_index[0].astype(jnp.int32)
    dst = edge_index[1].astype(jnp.int32)
    padlen = EPAD - E
    spread = jnp.arange(padlen, dtype=jnp.int32) % 128
    srcp = jnp.concatenate([src, spread]).reshape(NW * CPT, K)
    dstp = jnp.concatenate([dst, JUNK + spread]).reshape(NW * CPT, K)
    deg_parts = _sc_degree(dst)
    dis, xs = _tc_dis_scale(deg_parts, input_x)
    parts = _sc_spmm(xs, srcp, dstp)
    out = _tc_final(parts, xs, dis, W, b.reshape(1, F))
    return out


# speedup vs baseline: 1.9076x; 1.1875x over previous
import functools

import jax
import jax.numpy as jnp
from jax import lax
from jax.experimental import pallas as pl
from jax.experimental.pallas import tpu as pltpu
from jax.experimental.pallas import tpu_sc as plsc

N = 10000
E = 320000
F = 128
NC = 2
NS = 16
NW = NC * NS
K = 128
CPT = 80
EPT = CPT * K
EPAD = NW * EPT
JUNK = 10112
NPAD = 10240
RPT = NPAD // NS
ZR = 128
BR = 128
G = (N + BR - 1) // BR

_mesh = plsc.VectorSubcoreMesh(core_axis_name="c", subcore_axis_name="s")


K1 = 80
NCHUNK1 = 10000 // K1

@functools.partial(
    pl.kernel,
    out_type=jax.ShapeDtypeStruct((NC, NPAD, 16), jnp.float32),
    mesh=_mesh,
    scratch_types=[
        pltpu.VMEM_SHARED((NPAD, 16), jnp.float32),
        pltpu.VMEM((K1,), jnp.int32),
        pltpu.VMEM((K1, 16), jnp.float32),
        pltpu.VMEM((RPT, 16), jnp.float32),
    ],
)
def _sc_degree(dst_hbm, out_hbm, hist, idx_v, obuf, zbuf):
    cid = lax.axis_index("c")
    sid = lax.axis_index("s")
    wid = sid * NC + cid

    z16 = jnp.zeros((16,), jnp.float32)
    io = lax.iota(jnp.int32, 16)
    v1 = jnp.where(io == 0, 1.0, 0.0).astype(jnp.float32)

    def _zb(r, carry):
        zbuf[r] = z16
        return carry
    lax.fori_loop(0, RPT, _zb, 0)

    def _ob(r, carry):
        obuf[r] = v1
        return carry
    lax.fori_loop(0, K1, _ob, 0)

    pltpu.sync_copy(zbuf, hist.at[pl.ds(sid * RPT, RPT)])
    plsc.subcore_barrier()

    def _chunk(c, carry):
        pltpu.sync_copy(dst_hbm.at[pl.ds(wid * 10000 + c * K1, K1)], idx_v)
        pltpu.sync_copy(obuf, hist.at[idx_v], add=True)
        return carry
    lax.fori_loop(0, NCHUNK1, _chunk, 0)
    plsc.subcore_barrier()

    pltpu.sync_copy(hist.at[pl.ds(sid * RPT, RPT)], zbuf)
    pltpu.sync_copy(zbuf, out_hbm.at[cid, pl.ds(sid * RPT, RPT)])


@functools.partial(
    pl.kernel,
    out_type=jax.ShapeDtypeStruct((NC, NPAD, F), jnp.float32),
    mesh=_mesh,
    scratch_types=[
        pltpu.VMEM_SHARED((NPAD, F), jnp.float32),
        pltpu.VMEM((K,), jnp.int32),
        pltpu.VMEM((K,), jnp.int32),
        pltpu.VMEM((K,), jnp.int32),
        pltpu.VMEM((K,), jnp.int32),
        pltpu.VMEM((K, F), jnp.float32),
        pltpu.VMEM((ZR, F), jnp.float32),
        pltpu.SemaphoreType.DMA,
        pltpu.SemaphoreType.DMA,
        pltpu.SemaphoreType.DMA,
    ],
)
def _sc_spmm(xs_hbm, src_hbm, dst_hbm, out_hbm, acc,
             sidx0, sidx1, didx0, didx1, rows, zbuf, sem, is0, is1):
    cid = lax.axis_index("c")
    sid = lax.axis_index("s")
    wid = sid * NC + cid

    z16 = jnp.zeros((16,), jnp.float32)

    def _zb(r, carry):
        for j in range(F // 16):
            zbuf[r, pl.ds(j * 16, 16)] = z16
        return carry
    lax.fori_loop(0, ZR, _zb, 0)

    for t in range(RPT // ZR):
        pltpu.sync_copy(zbuf, acc.at[pl.ds(sid * RPT + t * ZR, ZR)])
    plsc.subcore_barrier()

    base = wid * CPT

    def _fire_idx(c, sb, db, s):
        pltpu.async_copy(src_hbm.at[base + c], sb, s)
        pltpu.async_copy(dst_hbm.at[base + c], db, s)

    def _wait_idx(c, sb, db, s):
        pltpu.make_async_copy(src_hbm.at[base + c], sb, s).wait()
        pltpu.make_async_copy(dst_hbm.at[base + c], db, s).wait()

    _fire_idx(0, sidx0, didx0, is0)

    def _pair(t, carry):
        c0 = 2 * t
        c1 = c0 + 1
        c2 = jnp.minimum(c0 + 2, CPT - 1)
        _wait_idx(c0, sidx0, didx0, is0)
        _fire_idx(c1, sidx1, didx1, is1)
        pltpu.async_copy(xs_hbm.at[sidx0], rows, sem).wait()
        pltpu.sync_copy(rows, acc.at[didx0], add=True)
        _wait_idx(c1, sidx1, didx1, is1)
        _fire_idx(c2, sidx0, didx0, is0)
        pltpu.async_copy(xs_hbm.at[sidx1], rows, sem).wait()
        pltpu.sync_copy(rows, acc.at[didx1], add=True)
        return carry
    lax.fori_loop(0, CPT // 2, _pair, 0)
    _wait_idx(CPT - 1, sidx0, didx0, is0)
    plsc.subcore_barrier()

    for t in range(RPT // ZR):
        pltpu.sync_copy(acc.at[pl.ds(sid * RPT + t * ZR, ZR)], zbuf)
        pltpu.sync_copy(zbuf, out_hbm.at[cid, pl.ds(sid * RPT + t * ZR, ZR)])


def _dis_scale_body(deg_ref, x_ref, dis_ref, xs_ref):
    p = deg_ref[...]
    deg = p[0, :, 0:1] + p[1, :, 0:1] + 1.0
    dis = lax.rsqrt(deg)
    dis_ref[...] = dis
    xs_ref[...] = x_ref[...] * dis


def _tc_dis_scale(deg_parts, x):
    return pl.pallas_call(
        _dis_scale_body,
        grid=(G,),
        in_specs=[
            pl.BlockSpec((NC, BR, 16), lambda i: (0, i, 0)),
            pl.BlockSpec((BR, F), lambda i: (i, 0)),
        ],
        out_specs=[
            pl.BlockSpec((BR, 1), lambda i: (i, 0)),
            pl.BlockSpec((BR, F), lambda i: (i, 0)),
        ],
        out_shape=[
            jax.ShapeDtypeStruct((G * BR, 1), jnp.float32),
            jax.ShapeDtypeStruct((N, F), jnp.float32),
        ],
    )(deg_parts, x)


def _final_body(p_ref, xs_ref, dis_ref, w_ref, b_ref, o_ref):
    p = p_ref[...]
    agg = (p[0] + p[1] + xs_ref[...]) * dis_ref[...]
    o_ref[...] = lax.dot_general(
        agg, w_ref[...], (((1,), (1,)), ((), ())),
        preferred_element_type=jnp.float32) + b_ref[...]


def _tc_final(parts, xs, dis, W, b2):
    return pl.pallas_call(
        _final_body,
        grid=(G,),
        in_specs=[
            pl.BlockSpec((NC, BR, F), lambda i: (0, i, 0)),
            pl.BlockSpec((BR, F), lambda i: (i, 0)),
            pl.BlockSpec((BR, 1), lambda i: (i, 0)),
            pl.BlockSpec((F, F), lambda i: (0, 0)),
            pl.BlockSpec((1, F), lambda i: (0, 0)),
        ],
        out_specs=pl.BlockSpec((BR, F), lambda i: (i, 0)),
        out_shape=jax.ShapeDtypeStruct((N, F), jnp.float32),
    )(parts, xs, dis, W, b2)


def kernel(input_x, edge_index, W, b):
    src = edge_index[0].astype(jnp.int32)
    dst = edge_index[1].astype(jnp.int32)
    padlen = EPAD - E
    spread = jnp.arange(padlen, dtype=jnp.int32) % 128
    srcp = jnp.concatenate([src, spread]).reshape(NW * CPT, K)
    dstp = jnp.concatenate([dst, JUNK + spread]).reshape(NW * CPT, K)
    deg_parts = _sc_degree(dst)
    dis, xs = _tc_dis_scale(deg_parts, input_x)
    parts = _sc_spmm(xs, srcp, dstp)
    out = _tc_final(parts, xs, dis, W, b.reshape(1, F))
    return out


# speedup vs baseline: 2.0526x; 1.0760x over previous
import functools

import jax
import jax.numpy as jnp
from jax import lax
from jax.experimental import pallas as pl
from jax.experimental.pallas import tpu as pltpu
from jax.experimental.pallas import tpu_sc as plsc

N = 10000
E = 320000
F = 128
NC = 2
NS = 16
NW = NC * NS
K = 128
CPT = 80
EPT = CPT * K
EPAD = NW * EPT
JUNK = 10112
NPAD = 10240
RPT = NPAD // NS
ZR = 128
BR = 128
G = (N + BR - 1) // BR

_mesh = plsc.VectorSubcoreMesh(core_axis_name="c", subcore_axis_name="s")


@functools.partial(
    pl.kernel,
    out_type=jax.ShapeDtypeStruct((NC, NPAD, 16), jnp.float32),
    mesh=_mesh,
    scratch_types=[
        pltpu.VMEM_SHARED((NPAD, 16), jnp.float32),
        pltpu.VMEM((K,), jnp.int32),
        pltpu.VMEM((K,), jnp.int32),
        pltpu.VMEM((K, 16), jnp.float32),
        pltpu.VMEM((RPT, 16), jnp.float32),
        pltpu.SemaphoreType.DMA,
        pltpu.SemaphoreType.DMA,
    ],
)
def _sc_degree(dst_hbm, out_hbm, hist, didx0, didx1, obuf, zbuf, is0, is1):
    cid = lax.axis_index("c")
    sid = lax.axis_index("s")
    wid = sid * NC + cid
    base = wid * CPT

    z16 = jnp.zeros((16,), jnp.float32)
    io = lax.iota(jnp.int32, 16)
    v1 = jnp.where(io == 0, 1.0, 0.0).astype(jnp.float32)

    def _zb(r, carry):
        zbuf[r] = z16
        return carry
    lax.fori_loop(0, RPT, _zb, 0)

    def _ob(r, carry):
        obuf[r] = v1
        return carry
    lax.fori_loop(0, K, _ob, 0)

    pltpu.sync_copy(zbuf, hist.at[pl.ds(sid * RPT, RPT)])
    plsc.subcore_barrier()

    pltpu.async_copy(dst_hbm.at[base], didx0, is0)

    def _pair(t, carry):
        c0 = 2 * t
        c1 = c0 + 1
        c2 = jnp.minimum(c0 + 2, CPT - 1)
        pltpu.make_async_copy(dst_hbm.at[base + c0], didx0, is0).wait()
        pltpu.async_copy(dst_hbm.at[base + c1], didx1, is1)
        pltpu.sync_copy(obuf, hist.at[didx0], add=True)
        pltpu.make_async_copy(dst_hbm.at[base + c1], didx1, is1).wait()
        pltpu.async_copy(dst_hbm.at[base + c2], didx0, is0)
        pltpu.sync_copy(obuf, hist.at[didx1], add=True)
        return carry
    lax.fori_loop(0, CPT // 2, _pair, 0)
    pltpu.make_async_copy(dst_hbm.at[base + CPT - 1], didx0, is0).wait()
    plsc.subcore_barrier()

    pltpu.sync_copy(hist.at[pl.ds(sid * RPT, RPT)], zbuf)
    pltpu.sync_copy(zbuf, out_hbm.at[cid, pl.ds(sid * RPT, RPT)])


@functools.partial(
    pl.kernel,
    out_type=jax.ShapeDtypeStruct((NC, NPAD, F), jnp.float32),
    mesh=_mesh,
    scratch_types=[
        pltpu.VMEM_SHARED((NPAD, F), jnp.float32),
        pltpu.VMEM((K,), jnp.int32),
        pltpu.VMEM((K,), jnp.int32),
        pltpu.VMEM((K,), jnp.int32),
        pltpu.VMEM((K,), jnp.int32),
        pltpu.VMEM((K, F), jnp.float32),
        pltpu.VMEM((ZR, F), jnp.float32),
        pltpu.SemaphoreType.DMA,
        pltpu.SemaphoreType.DMA,
        pltpu.SemaphoreType.DMA,
    ],
)
def _sc_spmm(xs_hbm, src_hbm, dst_hbm, out_hbm, acc,
             sidx0, sidx1, didx0, didx1, rows, zbuf, sem, is0, is1):
    cid = lax.axis_index("c")
    sid = lax.axis_index("s")
    wid = sid * NC + cid

    z16 = jnp.zeros((16,), jnp.float32)

    def _zb(r, carry):
        for j in range(F // 16):
            zbuf[r, pl.ds(j * 16, 16)] = z16
        return carry
    lax.fori_loop(0, ZR, _zb, 0)

    for t in range(RPT // ZR):
        pltpu.sync_copy(zbuf, acc.at[pl.ds(sid * RPT + t * ZR, ZR)])
    plsc.subcore_barrier()

    base = wid * CPT

    def _fire_idx(c, sb, db, s):
        pltpu.async_copy(src_hbm.at[base + c], sb, s)
        pltpu.async_copy(dst_hbm.at[base + c], db, s)

    def _wait_idx(c, sb, db, s):
        pltpu.make_async_copy(src_hbm.at[base + c], sb, s).wait()
        pltpu.make_async_copy(dst_hbm.at[base + c], db, s).wait()

    _fire_idx(0, sidx0, didx0, is0)

    def _pair(t, carry):
        c0 = 2 * t
        c1 = c0 + 1
        c2 = jnp.minimum(c0 + 2, CPT - 1)
        _wait_idx(c0, sidx0, didx0, is0)
        _fire_idx(c1, sidx1, didx1, is1)
        pltpu.async_copy(xs_hbm.at[sidx0], rows, sem).wait()
        pltpu.sync_copy(rows, acc.at[didx0], add=True)
        _wait_idx(c1, sidx1, didx1, is1)
        _fire_idx(c2, sidx0, didx0, is0)
        pltpu.async_copy(xs_hbm.at[sidx1], rows, sem).wait()
        pltpu.sync_copy(rows, acc.at[didx1], add=True)
        return carry
    lax.fori_loop(0, CPT // 2, _pair, 0)
    _wait_idx(CPT - 1, sidx0, didx0, is0)
    plsc.subcore_barrier()

    for t in range(RPT // ZR):
        pltpu.sync_copy(acc.at[pl.ds(sid * RPT + t * ZR, ZR)], zbuf)
        pltpu.sync_copy(zbuf, out_hbm.at[cid, pl.ds(sid * RPT + t * ZR, ZR)])


def _dis_scale_body(deg_ref, x_ref, dis_ref, xs_ref):
    p = deg_ref[...]
    deg = p[0, :, 0:1] + p[1, :, 0:1] + 1.0
    dis = lax.rsqrt(deg)
    dis_ref[...] = dis
    xs_ref[...] = x_ref[...] * dis


def _tc_dis_scale(deg_parts, x):
    return pl.pallas_call(
        _dis_scale_body,
        grid=(G,),
        in_specs=[
            pl.BlockSpec((NC, BR, 16), lambda i: (0, i, 0)),
            pl.BlockSpec((BR, F), lambda i: (i, 0)),
        ],
        out_specs=[
            pl.BlockSpec((BR, 1), lambda i: (i, 0)),
            pl.BlockSpec((BR, F), lambda i: (i, 0)),
        ],
        out_shape=[
            jax.ShapeDtypeStruct((G * BR, 1), jnp.float32),
            jax.ShapeDtypeStruct((N, F), jnp.float32),
        ],
    )(deg_parts, x)


def _final_body(p_ref, xs_ref, dis_ref, w_ref, b_ref, o_ref):
    p = p_ref[...]
    agg = (p[0] + p[1] + xs_ref[...]) * dis_ref[...]
    o_ref[...] = lax.dot_general(
        agg, w_ref[...], (((1,), (1,)), ((), ())),
        preferred_element_type=jnp.float32) + b_ref[...]


def _tc_final(parts, xs, dis, W, b2):
    return pl.pallas_call(
        _final_body,
        grid=(G,),
        in_specs=[
            pl.BlockSpec((NC, BR, F), lambda i: (0, i, 0)),
            pl.BlockSpec((BR, F), lambda i: (i, 0)),
            pl.BlockSpec((BR, 1), lambda i: (i, 0)),
            pl.BlockSpec((F, F), lambda i: (0, 0)),
            pl.BlockSpec((1, F), lambda i: (0, 0)),
        ],
        out_specs=pl.BlockSpec((BR, F), lambda i: (i, 0)),
        out_shape=jax.ShapeDtypeStruct((N, F), jnp.float32),
    )(parts, xs, dis, W, b2)


def kernel(input_x, edge_index, W, b):
    src = edge_index[0].astype(jnp.int32)
    dst = edge_index[1].astype(jnp.int32)
    padlen = EPAD - E
    spread = jnp.arange(padlen, dtype=jnp.int32) % 128
    srcp = jnp.concatenate([src, spread]).reshape(NW * CPT, K)
    dstp = jnp.concatenate([dst, JUNK + spread]).reshape(NW * CPT, K)
    deg_parts = _sc_degree(dstp)
    dis, xs = _tc_dis_scale(deg_parts, input_x)
    parts = _sc_spmm(xs, srcp, dstp)
    out = _tc_final(parts, xs, dis, W, b.reshape(1, F))
    return out
